# Initial kernel scaffold; baseline (speedup 1.0000x reference)
import jax
import jax.numpy as jnp
from jax.experimental import pallas as pl


def kernel(x, edge_index, W1, b1, W2, b2, actor_W, actor_b, critic_W, critic_b):
    raise NotImplementedError("write your pallas kernel here")


# speedup vs baseline: 5.4945x; 5.4945x over previous
import functools

import jax
import jax.numpy as jnp
from jax import lax
from jax.experimental import pallas as pl
from jax.experimental.pallas import tpu as pltpu
from jax.experimental.pallas import tpu_sc as plsc

NN = 1024
NE = 65536
NW = 32
EPW = NE // NW
NEG_INF = float("-inf")


def _wid():
    return lax.axis_index("s") * 2 + lax.axis_index("c")


def _iota16():
    return lax.iota(jnp.int32, 16)


def _sc_hist_body(dst_hbm, out_hbm, dst_v, cnt_v):
    w = _wid()

    def zero_body(i, _):
        cnt_v[pl.ds(i * 16, 16)] = jnp.zeros((16,), jnp.float32)
        return 0

    lax.fori_loop(0, NN // 16, zero_body, 0)
    pltpu.sync_copy(dst_hbm.at[pl.ds(w * EPW, EPW)], dst_v)
    ones = jnp.ones((16,), jnp.float32)

    def body(g, _):
        d16 = dst_v[pl.ds(g * 16, 16)]
        plsc.addupdate_scatter(cnt_v, [d16], ones)
        return 0

    lax.fori_loop(0, EPW // 16, body, 0)
    pltpu.sync_copy(cnt_v, out_hbm.at[w])


def _sc_hist(dst):
    mesh = plsc.VectorSubcoreMesh(core_axis_name="c", subcore_axis_name="s")
    f = pl.kernel(
        _sc_hist_body,
        out_type=jax.ShapeDtypeStruct((NW, NN), jnp.float32),
        mesh=mesh,
        compiler_params=pltpu.CompilerParams(needs_layout_passes=False, use_tc_tiling_on_sc=False),
        scratch_types=[
            pltpu.VMEM((EPW,), jnp.int32),
            pltpu.VMEM((NN,), jnp.float32),
        ],
    )
    return f(dst)


def _tc_prep1_body(x_ref, w1_ref, degp_ref, xw_ref, dinv_ref):
    deg = jnp.sum(degp_ref[...], axis=0) + 1.0
    dinv_ref[...] = jnp.where(deg > 0, lax.rsqrt(deg), 0.0)
    xw_ref[...] = lax.dot_general(
        x_ref[...], w1_ref[...], (((1,), (1,)), ((), ())),
        preferred_element_type=jnp.float32)


def _tc_prep1(x, W1, degp):
    return pl.pallas_call(
        _tc_prep1_body,
        out_shape=(
            jax.ShapeDtypeStruct((NN, 32), jnp.float32),
            jax.ShapeDtypeStruct((NN,), jnp.float32),
        ),
    )(x, W1, degp)


def _sc_layer_body(F, first, *refs):
    if first:
        (src_hbm, dst_hbm, dinv_hbm, xw_hbm, out_hbm, norm_hbm,
         src_v, dst_v, nrm_v, dinv_v, xw_v, acc_v) = refs
    else:
        (src_hbm, dst_hbm, norm_hbm, dinv_hbm, xw_hbm, out_hbm,
         src_v, dst_v, nrm_v, dinv_v, xw_v, acc_v) = refs
    w = _wid()
    base = w * EPW
    pltpu.sync_copy(src_hbm.at[pl.ds(base, EPW)], src_v)
    pltpu.sync_copy(dst_hbm.at[pl.ds(base, EPW)], dst_v)
    pltpu.sync_copy(dinv_hbm, dinv_v)
    pltpu.sync_copy(xw_hbm, xw_v)
    if not first:
        pltpu.sync_copy(norm_hbm.at[pl.ds(base, EPW)], nrm_v)

    def zero_body(i, _):
        acc_v[i, pl.ds(0, 16)] = jnp.zeros((16,), jnp.float32)
        if F > 16:
            acc_v[i, pl.ds(F - 16, 16)] = jnp.zeros((16,), jnp.float32)
        return 0

    lax.fori_loop(0, NN, zero_body, 0)

    def edge_body(g, _):
        s16 = src_v[pl.ds(g * 16, 16)]
        d16 = dst_v[pl.ds(g * 16, 16)]
        if first:
            n16 = (plsc.load_gather(dinv_v, [s16])
                   * plsc.load_gather(dinv_v, [d16]))
            nrm_v[pl.ds(g * 16, 16)] = n16
        else:
            n16 = nrm_v[pl.ds(g * 16, 16)]
        for c in range(F):
            c16 = jnp.full((16,), c, jnp.int32)
            v = plsc.load_gather(xw_v, [s16, c16])
            plsc.addupdate_scatter(acc_v, [d16, c16], v * n16)
        return 0

    lax.fori_loop(0, EPW // 16, edge_body, 0)

    for t in range(2):
        i16 = w * 32 + t * 16 + _iota16()
        dv = plsc.load_gather(dinv_v, [i16])
        n16 = dv * dv
        for c in range(F):
            c16 = jnp.full((16,), c, jnp.int32)
            v = plsc.load_gather(xw_v, [i16, c16])
            plsc.addupdate_scatter(acc_v, [i16, c16], v * n16)

    if first:
        pltpu.sync_copy(nrm_v, norm_hbm.at[pl.ds(base, EPW)])
    pltpu.sync_copy(acc_v, out_hbm.at[w])


def _sc_layer(src, dst, dinv, xw, norm=None):
    F = xw.shape[1]
    first = norm is None
    mesh = plsc.VectorSubcoreMesh(core_axis_name="c", subcore_axis_name="s")
    out_type = [jax.ShapeDtypeStruct((NW, NN, F), jnp.float32)]
    if first:
        out_type.append(jax.ShapeDtypeStruct((NE,), jnp.float32))
    f = pl.kernel(
        functools.partial(_sc_layer_body, F, first),
        out_type=tuple(out_type),
        mesh=mesh,
        compiler_params=pltpu.CompilerParams(needs_layout_passes=False, use_tc_tiling_on_sc=False),
        scratch_types=[
            pltpu.VMEM((EPW,), jnp.int32),
            pltpu.VMEM((EPW,), jnp.int32),
            pltpu.VMEM((EPW,), jnp.float32),
            pltpu.VMEM((NN,), jnp.float32),
            pltpu.VMEM((NN, F), jnp.float32),
            pltpu.VMEM((NN, F), jnp.float32),
        ],
    )
    if first:
        return f(src, dst, dinv, xw)
    return f(src, dst, norm, dinv, xw)


def _tc_prep2_body(part_ref, b1_ref, w2_ref, xw2_ref):
    h1 = jnp.maximum(jnp.sum(part_ref[...], axis=0) + b1_ref[...][None, :], 0.0)
    h1b = h1.astype(jnp.bfloat16).astype(jnp.float32)
    xw2_ref[...] = lax.dot_general(
        h1b, w2_ref[...], (((1,), (1,)), ((), ())),
        preferred_element_type=jnp.float32)


def _tc_prep2(part1, b1, W2):
    return pl.pallas_call(
        _tc_prep2_body,
        out_shape=jax.ShapeDtypeStruct((NN, 20), jnp.float32),
    )(part1, b1, W2)


def _tc_prep3_body(part_ref, b2_ref, cw_ref, cb_ref, xa_ref, crit_ref):
    h2 = jnp.maximum(jnp.sum(part_ref[...], axis=0) + b2_ref[...][None, :], 0.0)
    xa_ref[...] = jnp.sum(h2, axis=0, keepdims=True) / 1024.0
    h2b = h2.astype(jnp.bfloat16).astype(jnp.float32)
    colsum = jnp.sum(h2b, axis=0, keepdims=True)
    total = jnp.sum(colsum * cw_ref[...])
    crit_ref[...] = jnp.broadcast_to(total / 1024.0 + cb_ref[0], (1, 1))


def _tc_prep3(part2, b2, critic_W, critic_b):
    return pl.pallas_call(
        _tc_prep3_body,
        out_shape=(
            jax.ShapeDtypeStruct((1, 20), jnp.float32),
            jax.ShapeDtypeStruct((1, 1), jnp.float32),
        ),
    )(part2, b2, critic_W, critic_b)


_BR = 8


def _tc_actor_body(xa_ref, aw_ref, ab_ref, probs_ref, winp_ref, winc_ref):
    l = lax.dot_general(
        xa_ref[...], aw_ref[...], (((1,), (0,)), ((), ())),
        preferred_element_type=jnp.float32)
    acc = jnp.reshape(l, (_BR, NN)) + ab_ref[...]
    m = jnp.max(acc, axis=1, keepdims=True)
    e = jnp.exp(acc - m)
    s = jnp.sum(e, axis=1, keepdims=True)
    p = e / s
    probs_ref[...] = p
    wp = jnp.max(p, axis=1, keepdims=True)
    winp_ref[...] = wp
    cols = lax.broadcasted_iota(jnp.int32, (_BR, NN), 1)
    winc_ref[...] = jnp.min(
        jnp.where(p == wp, cols, jnp.int32(NN * NN)), axis=1, keepdims=True)


def _tc_actor(xa, aw3, ab2):
    grid = (NN // _BR,)
    return pl.pallas_call(
        _tc_actor_body,
        grid=grid,
        in_specs=[
            pl.BlockSpec((1, 20), lambda i: (0, 0)),
            pl.BlockSpec((20, _BR * NN), lambda i: (0, i)),
            pl.BlockSpec((_BR, NN), lambda i: (i, 0)),
        ],
        out_specs=[
            pl.BlockSpec((_BR, NN), lambda i: (i, 0)),
            pl.BlockSpec((_BR, 1), lambda i: (i, 0)),
            pl.BlockSpec((_BR, 1), lambda i: (i, 0)),
        ],
        out_shape=(
            jax.ShapeDtypeStruct((NN, NN), jnp.float32),
            jax.ShapeDtypeStruct((NN, 1), jnp.float32),
            jax.ShapeDtypeStruct((NN, 1), jnp.int32),
        ),
    )(xa, aw3, ab2)


def _tc_select_body(winp_ref, winc_ref, sel_ref):
    wp = winp_ref[...]
    rows = lax.broadcasted_iota(jnp.int32, (NN, 1), 0)
    maxv = jnp.max(wp)
    mi = jnp.min(jnp.where(wp == maxv, rows, jnp.int32(NN * NN)))
    mj = jnp.sum(jnp.where(rows == mi, winc_ref[...], 0))
    k = lax.broadcasted_iota(jnp.int32, (1, 4), 1)
    sel_ref[...] = jnp.where(k == 0, mi, jnp.where(k == 1, mj, 0))


def _tc_select(winp, winc):
    return pl.pallas_call(
        _tc_select_body,
        out_shape=jax.ShapeDtypeStruct((1, 4), jnp.int32),
    )(winp, winc)


def _tc_fin_body(sel_ref, winp_ref, winc_ref, prow_ref, p0_ref,
                 act_ref, lp_ref):
    mi = sel_ref[0]
    mj = sel_ref[1]
    big = jnp.int32(NN * NN)

    rows = lax.broadcasted_iota(jnp.int32, (NN, 1), 0)
    rmask = rows < mi
    rvals = jnp.where(rmask, winp_ref[...], NEG_INF)
    rbest = jnp.max(rvals)
    rrow = jnp.min(jnp.where(rvals == rbest, rows, big))
    rcol = jnp.sum(jnp.where(rows == rrow, winc_ref[...], 0))

    cols = lax.broadcasted_iota(jnp.int32, (1, NN), 1)
    r8 = lax.broadcasted_iota(jnp.int32, (8, NN), 0)
    pmi = jnp.sum(
        jnp.where(r8 == mi % 8, prow_ref[...], 0.0), axis=0,
        keepdims=True)
    mvals = jnp.where(cols < mj, pmi, NEG_INF)
    mbest = jnp.max(mvals)
    mcol = jnp.min(jnp.where(mvals == mbest, cols, big))

    use_rows = rbest >= mbest
    any_prefix = (rbest > NEG_INF) | (mbest > NEG_INF)
    idx_s = jnp.where(
        use_rows, rrow * NN + rcol, mi * NN + mcol)
    idx_s = jnp.where(any_prefix, idx_s, 0)
    idx_max = mi * NN + mj
    has_prefix = idx_max > 0
    si = jnp.where(has_prefix, idx_s // NN, 0)
    sj = jnp.where(has_prefix, idx_s % NN, 0)

    r2 = lax.broadcasted_iota(jnp.int32, (2, 2), 0)
    c2 = lax.broadcasted_iota(jnp.int32, (2, 2), 1)
    act = jnp.where(
        r2 == 0,
        jnp.where(c2 == 0, mi, mj),
        jnp.where(c2 == 0, si, sj))
    act_ref[...] = act

    p0 = jnp.sum(
        jnp.where(r8 == 0, p0_ref[...], 0.0), axis=0,
        keepdims=True)

    def gat(idx):
        return jnp.sum(jnp.where(cols == idx, p0, 0.0))

    lp = jnp.where(
        r2 == 0,
        jnp.where(c2 == 0, gat(mi), gat(mj)),
        jnp.where(c2 == 0, gat(si), gat(sj)))
    lp_ref[...] = -jnp.log(lp)


def _tc_finalize(sel, winp, winc, probs):
    grid_spec = pltpu.PrefetchScalarGridSpec(
        num_scalar_prefetch=1,
        grid=(1,),
        in_specs=[
            pl.BlockSpec((NN, 1), lambda i, s: (0, 0)),
            pl.BlockSpec((NN, 1), lambda i, s: (0, 0)),
            pl.BlockSpec((8, NN), lambda i, s: (s[0] // 8, 0)),
            pl.BlockSpec((8, NN), lambda i, s: (0, 0)),
        ],
        out_specs=[
            pl.BlockSpec((2, 2), lambda i, s: (0, 0)),
            pl.BlockSpec((2, 2), lambda i, s: (0, 0)),
        ],
    )
    return pl.pallas_call(
        _tc_fin_body,
        grid_spec=grid_spec,
        out_shape=(
            jax.ShapeDtypeStruct((2, 2), jnp.int32),
            jax.ShapeDtypeStruct((2, 2), jnp.float32),
        ),
    )(sel, winp, winc, probs, probs)


def kernel(x, edge_index, W1, b1, W2, b2, actor_W, actor_b, critic_W,
           critic_b):
    src = edge_index[0]
    dst = edge_index[1]

    degp = _sc_hist(dst)
    xw1, dinv = _tc_prep1(x, W1, degp)
    part1, norm = _sc_layer(src, dst, dinv, xw1)
    xw2 = _tc_prep2(part1, b1, W2)
    part2, = _sc_layer(src, dst, dinv, xw2, norm=norm)
    xa, crit = _tc_prep3(part2, b2, critic_W, critic_b)

    aw_t = jnp.transpose(actor_W)
    ab2 = actor_b.reshape(NN, NN)
    probs, winp, winc = _tc_actor(xa, aw_t, ab2)
    sel = _tc_select(winp, winc)
    actions, log_probs = _tc_finalize(sel.reshape(4), winp, winc, probs)
    critic = crit.reshape(1)
    return actions, log_probs, critic


# speedup vs baseline: 8.5902x; 1.5634x over previous
import functools

import jax
import jax.numpy as jnp
from jax import lax
from jax.experimental import pallas as pl
from jax.experimental.pallas import tpu as pltpu
from jax.experimental.pallas import tpu_sc as plsc

NN = 1024
NE = 65536
NW = 32
ROWS = NN // NW
ECH = 8192
NEG_INF = float("-inf")


def _wid():
    return lax.axis_index("s") * 2 + lax.axis_index("c")


def _sc_abuild_body(src_hbm, dst_hbm, out_hbm, src_v, dst_v, acc_v):
    w = _wid()
    lo = w * ROWS

    def zero_body(i, _):
        for j in range(NN // 16):
            acc_v[i, pl.ds(j * 16, 16)] = jnp.zeros((16,), jnp.float32)
        return 0

    lax.fori_loop(0, ROWS, zero_body, 0)

    ones = jnp.ones((16,), jnp.float32)
    for k in range(NE // ECH):
        pltpu.sync_copy(src_hbm.at[pl.ds(k * ECH, ECH)], src_v)
        pltpu.sync_copy(dst_hbm.at[pl.ds(k * ECH, ECH)], dst_v)

        def group_body(g, _):
            s16 = src_v[pl.ds(g * 16, 16)]
            d16 = dst_v[pl.ds(g * 16, 16)]
            local = d16 - lo
            mask = (local >= 0) & (local < ROWS)
            row = jnp.where(mask, local, 0)
            plsc.addupdate_scatter(acc_v, [row, s16], ones, mask=mask)
            return 0

        lax.fori_loop(0, ECH // 16, group_body, 0)

    pltpu.sync_copy(acc_v, out_hbm.at[pl.ds(lo, ROWS)])


def _sc_abuild(src, dst):
    mesh = plsc.VectorSubcoreMesh(core_axis_name="c", subcore_axis_name="s")
    f = pl.kernel(
        _sc_abuild_body,
        out_type=jax.ShapeDtypeStruct((NN, NN), jnp.float32),
        mesh=mesh,
        compiler_params=pltpu.CompilerParams(
            needs_layout_passes=False, use_tc_tiling_on_sc=False),
        scratch_types=[
            pltpu.VMEM((ECH,), jnp.int32),
            pltpu.VMEM((ECH,), jnp.int32),
            pltpu.VMEM((ROWS, NN), jnp.float32),
        ],
    )
    return f(src, dst)


def _tc_gcn_body(a_ref, x_ref, w1_ref, b1_ref, w2_ref, b2_ref, cw_ref,
                 cb_ref, xa_ref, crit_ref):
    A = a_ref[...]
    deg = jnp.sum(A, axis=1, keepdims=True) + 1.0
    dinv = lax.rsqrt(deg)

    xw1 = lax.dot_general(
        x_ref[...], w1_ref[...], (((1,), (1,)), ((), ())),
        preferred_element_type=jnp.float32)
    t1 = dinv * xw1
    a1 = lax.dot_general(
        A, t1, (((1,), (0,)), ((), ())), preferred_element_type=jnp.float32)
    h1 = jnp.maximum(dinv * (a1 + t1) + b1_ref[...][None, :], 0.0)

    h1b = h1.astype(jnp.bfloat16).astype(jnp.float32)
    xw2 = lax.dot_general(
        h1b, w2_ref[...], (((1,), (1,)), ((), ())),
        preferred_element_type=jnp.float32)
    t2 = dinv * xw2
    a2 = lax.dot_general(
        A, t2, (((1,), (0,)), ((), ())), preferred_element_type=jnp.float32)
    h2 = jnp.maximum(dinv * (a2 + t2) + b2_ref[...][None, :], 0.0)

    xa_ref[...] = jnp.sum(h2, axis=0, keepdims=True) / 1024.0
    h2b = h2.astype(jnp.bfloat16).astype(jnp.float32)
    colsum = jnp.sum(h2b, axis=0, keepdims=True)
    total = jnp.sum(colsum * cw_ref[...])
    crit_ref[...] = jnp.broadcast_to(total / 1024.0 + cb_ref[0], (1, 1))


def _tc_gcn(A, x, W1, b1, W2, b2, critic_W, critic_b):
    return pl.pallas_call(
        _tc_gcn_body,
        out_shape=(
            jax.ShapeDtypeStruct((1, 20), jnp.float32),
            jax.ShapeDtypeStruct((1, 1), jnp.float32),
        ),
    )(A, x, W1, b1, W2, b2, critic_W, critic_b)


_BR = 8


def _tc_actor_body(xa_ref, aw_ref, ab_ref, probs_ref, winp_ref, winc_ref):
    l = lax.dot_general(
        xa_ref[...], aw_ref[...], (((1,), (0,)), ((), ())),
        preferred_element_type=jnp.float32)
    acc = jnp.reshape(l, (_BR, NN)) + ab_ref[...]
    m = jnp.max(acc, axis=1, keepdims=True)
    e = jnp.exp(acc - m)
    s = jnp.sum(e, axis=1, keepdims=True)
    p = e / s
    probs_ref[...] = p
    wp = jnp.max(p, axis=1, keepdims=True)
    winp_ref[...] = wp
    cols = lax.broadcasted_iota(jnp.int32, (_BR, NN), 1)
    winc_ref[...] = jnp.min(
        jnp.where(p == wp, cols, jnp.int32(NN * NN)), axis=1, keepdims=True)


def _tc_actor(xa, aw3, ab2):
    grid = (NN // _BR,)
    return pl.pallas_call(
        _tc_actor_body,
        grid=grid,
        in_specs=[
            pl.BlockSpec((1, 20), lambda i: (0, 0)),
            pl.BlockSpec((20, _BR * NN), lambda i: (0, i)),
            pl.BlockSpec((_BR, NN), lambda i: (i, 0)),
        ],
        out_specs=[
            pl.BlockSpec((_BR, NN), lambda i: (i, 0)),
            pl.BlockSpec((_BR, 1), lambda i: (i, 0)),
            pl.BlockSpec((_BR, 1), lambda i: (i, 0)),
        ],
        out_shape=(
            jax.ShapeDtypeStruct((NN, NN), jnp.float32),
            jax.ShapeDtypeStruct((NN, 1), jnp.float32),
            jax.ShapeDtypeStruct((NN, 1), jnp.int32),
        ),
    )(xa, aw3, ab2)


def _tc_select_body(winp_ref, winc_ref, sel_ref):
    wp = winp_ref[...]
    rows = lax.broadcasted_iota(jnp.int32, (NN, 1), 0)
    maxv = jnp.max(wp)
    mi = jnp.min(jnp.where(wp == maxv, rows, jnp.int32(NN * NN)))
    mj = jnp.sum(jnp.where(rows == mi, winc_ref[...], 0))
    k = lax.broadcasted_iota(jnp.int32, (1, 4), 1)
    sel_ref[...] = jnp.where(k == 0, mi, jnp.where(k == 1, mj, 0))


def _tc_select(winp, winc):
    return pl.pallas_call(
        _tc_select_body,
        out_shape=jax.ShapeDtypeStruct((1, 4), jnp.int32),
    )(winp, winc)


def _tc_fin_body(sel_ref, winp_ref, winc_ref, prow_ref, p0_ref,
                 act_ref, lp_ref):
    mi = sel_ref[0]
    mj = sel_ref[1]
    big = jnp.int32(NN * NN)

    rows = lax.broadcasted_iota(jnp.int32, (NN, 1), 0)
    rmask = rows < mi
    rvals = jnp.where(rmask, winp_ref[...], NEG_INF)
    rbest = jnp.max(rvals)
    rrow = jnp.min(jnp.where(rvals == rbest, rows, big))
    rcol = jnp.sum(jnp.where(rows == rrow, winc_ref[...], 0))

    cols = lax.broadcasted_iota(jnp.int32, (1, NN), 1)
    r8 = lax.broadcasted_iota(jnp.int32, (8, NN), 0)
    pmi = jnp.sum(
        jnp.where(r8 == mi % 8, prow_ref[...], 0.0), axis=0,
        keepdims=True)
    mvals = jnp.where(cols < mj, pmi, NEG_INF)
    mbest = jnp.max(mvals)
    mcol = jnp.min(jnp.where(mvals == mbest, cols, big))

    use_rows = rbest >= mbest
    any_prefix = (rbest > NEG_INF) | (mbest > NEG_INF)
    idx_s = jnp.where(
        use_rows, rrow * NN + rcol, mi * NN + mcol)
    idx_s = jnp.where(any_prefix, idx_s, 0)
    idx_max = mi * NN + mj
    has_prefix = idx_max > 0
    si = jnp.where(has_prefix, idx_s // NN, 0)
    sj = jnp.where(has_prefix, idx_s % NN, 0)

    r2 = lax.broadcasted_iota(jnp.int32, (2, 2), 0)
    c2 = lax.broadcasted_iota(jnp.int32, (2, 2), 1)
    act = jnp.where(
        r2 == 0,
        jnp.where(c2 == 0, mi, mj),
        jnp.where(c2 == 0, si, sj))
    act_ref[...] = act

    p0 = jnp.sum(
        jnp.where(r8 == 0, p0_ref[...], 0.0), axis=0,
        keepdims=True)

    def gat(idx):
        return jnp.sum(jnp.where(cols == idx, p0, 0.0))

    lp = jnp.where(
        r2 == 0,
        jnp.where(c2 == 0, gat(mi), gat(mj)),
        jnp.where(c2 == 0, gat(si), gat(sj)))
    lp_ref[...] = -jnp.log(lp)


def _tc_finalize(sel, winp, winc, probs):
    grid_spec = pltpu.PrefetchScalarGridSpec(
        num_scalar_prefetch=1,
        grid=(1,),
        in_specs=[
            pl.BlockSpec((NN, 1), lambda i, s: (0, 0)),
            pl.BlockSpec((NN, 1), lambda i, s: (0, 0)),
            pl.BlockSpec((8, NN), lambda i, s: (s[0] // 8, 0)),
            pl.BlockSpec((8, NN), lambda i, s: (0, 0)),
        ],
        out_specs=[
            pl.BlockSpec((2, 2), lambda i, s: (0, 0)),
            pl.BlockSpec((2, 2), lambda i, s: (0, 0)),
        ],
    )
    return pl.pallas_call(
        _tc_fin_body,
        grid_spec=grid_spec,
        out_shape=(
            jax.ShapeDtypeStruct((2, 2), jnp.int32),
            jax.ShapeDtypeStruct((2, 2), jnp.float32),
        ),
    )(sel, winp, winc, probs, probs)


def kernel(x, edge_index, W1, b1, W2, b2, actor_W, actor_b, critic_W,
           critic_b):
    src = edge_index[0]
    dst = edge_index[1]

    A = _sc_abuild(src, dst)
    xa, crit = _tc_gcn(A, x, W1, b1, W2, b2, critic_W, critic_b)

    aw_t = jnp.transpose(actor_W)
    ab2 = actor_b.reshape(NN, NN)
    probs, winp, winc = _tc_actor(xa, aw_t, ab2)
    sel = _tc_select(winp, winc)
    actions, log_probs = _tc_finalize(sel.reshape(4), winp, winc, probs)
    critic = crit.reshape(1)
    return actions, log_probs, critic


# speedup vs baseline: 14.0840x; 1.6395x over previous
import functools

import jax
import jax.numpy as jnp
from jax import lax
from jax.experimental import pallas as pl
from jax.experimental.pallas import tpu as pltpu
from jax.experimental.pallas import tpu_sc as plsc

NN = 1024
NE = 65536
NW = 32
ROWS = NN // NW
ECH = 8192
NEG_INF = float("-inf")


def _wid():
    return lax.axis_index("s") * 2 + lax.axis_index("c")


def _sc_abuild_body(src_hbm, dst_hbm, out_hbm, src_v, dst_v, acc_v):
    w = _wid()
    lo = w * ROWS

    def zero_body(i, _):
        for j in range(NN // 16):
            acc_v[i, pl.ds(j * 16, 16)] = jnp.zeros((16,), jnp.float32)
        return 0

    lax.fori_loop(0, ROWS, zero_body, 0)

    ones = jnp.ones((16,), jnp.float32)
    for k in range(NE // ECH):
        pltpu.sync_copy(src_hbm.at[pl.ds(k * ECH, ECH)], src_v)
        pltpu.sync_copy(dst_hbm.at[pl.ds(k * ECH, ECH)], dst_v)

        def group_body(g, _):
            for u in range(4):
                off = (g * 4 + u) * 16
                s16 = src_v[pl.ds(off, 16)]
                d16 = dst_v[pl.ds(off, 16)]
                local = d16 - lo
                mask = local.astype(jnp.uint32) < ROWS
                plsc.addupdate_scatter(acc_v, [local, s16], ones, mask=mask)
            return 0

        lax.fori_loop(0, ECH // 64, group_body, 0)

    pltpu.sync_copy(acc_v, out_hbm.at[pl.ds(lo, ROWS)])


def _sc_abuild(src, dst):
    mesh = plsc.VectorSubcoreMesh(core_axis_name="c", subcore_axis_name="s")
    f = pl.kernel(
        _sc_abuild_body,
        out_type=jax.ShapeDtypeStruct((NN, NN), jnp.float32),
        mesh=mesh,
        compiler_params=pltpu.CompilerParams(
            needs_layout_passes=False, use_tc_tiling_on_sc=False),
        scratch_types=[
            pltpu.VMEM((ECH,), jnp.int32),
            pltpu.VMEM((ECH,), jnp.int32),
            pltpu.VMEM((ROWS, NN), jnp.float32),
        ],
    )
    return f(src, dst)


def _tc_gcn_body(a_ref, x_ref, w1_ref, b1_ref, w2_ref, b2_ref, cw_ref,
                 cb_ref, xa_ref, crit_ref):
    A = a_ref[...]
    deg = jnp.sum(A, axis=1, keepdims=True) + 1.0
    dinv = lax.rsqrt(deg)

    xw1 = lax.dot_general(
        x_ref[...], w1_ref[...], (((1,), (1,)), ((), ())),
        preferred_element_type=jnp.float32)
    t1 = dinv * xw1
    a1 = lax.dot_general(
        A, t1, (((1,), (0,)), ((), ())), preferred_element_type=jnp.float32)
    h1 = jnp.maximum(dinv * (a1 + t1) + b1_ref[...][None, :], 0.0)

    h1b = h1.astype(jnp.bfloat16).astype(jnp.float32)
    xw2 = lax.dot_general(
        h1b, w2_ref[...], (((1,), (1,)), ((), ())),
        preferred_element_type=jnp.float32)
    t2 = dinv * xw2
    a2 = lax.dot_general(
        A, t2, (((1,), (0,)), ((), ())), preferred_element_type=jnp.float32)
    h2 = jnp.maximum(dinv * (a2 + t2) + b2_ref[...][None, :], 0.0)

    xa_ref[...] = jnp.sum(h2, axis=0, keepdims=True) / 1024.0
    h2b = h2.astype(jnp.bfloat16).astype(jnp.float32)
    colsum = jnp.sum(h2b, axis=0, keepdims=True)
    total = jnp.sum(colsum * cw_ref[...])
    crit_ref[...] = jnp.broadcast_to(total / 1024.0 + cb_ref[0], (1, 1))


def _tc_gcn(A, x, W1, b1, W2, b2, critic_W, critic_b):
    return pl.pallas_call(
        _tc_gcn_body,
        out_shape=(
            jax.ShapeDtypeStruct((1, 20), jnp.float32),
            jax.ShapeDtypeStruct((1, 1), jnp.float32),
        ),
    )(A, x, W1, b1, W2, b2, critic_W, critic_b)


_BR = 64


def _tc_actor_body(xa_ref, aw_ref, ab_ref, probs_ref, winp_ref, winc_ref):
    l = lax.dot_general(
        xa_ref[...], aw_ref[...], (((1,), (0,)), ((), ())),
        preferred_element_type=jnp.float32)
    acc = jnp.reshape(l, (_BR, NN)) + ab_ref[...]
    m = jnp.max(acc, axis=1, keepdims=True)
    e = jnp.exp(acc - m)
    s = jnp.sum(e, axis=1, keepdims=True)
    p = e / s
    probs_ref[...] = p
    wp = jnp.max(p, axis=1, keepdims=True)
    winp_ref[...] = wp
    cols = lax.broadcasted_iota(jnp.int32, (_BR, NN), 1)
    winc_ref[...] = jnp.min(
        jnp.where(p == wp, cols, jnp.int32(NN * NN)), axis=1, keepdims=True)


def _tc_actor(xa, aw3, ab2):
    grid = (NN // _BR,)
    return pl.pallas_call(
        _tc_actor_body,
        grid=grid,
        in_specs=[
            pl.BlockSpec((1, 20), lambda i: (0, 0)),
            pl.BlockSpec((20, _BR * NN), lambda i: (0, i)),
            pl.BlockSpec((_BR, NN), lambda i: (i, 0)),
        ],
        out_specs=[
            pl.BlockSpec((_BR, NN), lambda i: (i, 0)),
            pl.BlockSpec((_BR, 1), lambda i: (i, 0)),
            pl.BlockSpec((_BR, 1), lambda i: (i, 0)),
        ],
        out_shape=(
            jax.ShapeDtypeStruct((NN, NN), jnp.float32),
            jax.ShapeDtypeStruct((NN, 1), jnp.float32),
            jax.ShapeDtypeStruct((NN, 1), jnp.int32),
        ),
    )(xa, aw3, ab2)


def _tc_select_body(winp_ref, winc_ref, sel_ref):
    wp = winp_ref[...]
    rows = lax.broadcasted_iota(jnp.int32, (NN, 1), 0)
    maxv = jnp.max(wp)
    mi = jnp.min(jnp.where(wp == maxv, rows, jnp.int32(NN * NN)))
    mj = jnp.sum(jnp.where(rows == mi, winc_ref[...], 0))
    k = lax.broadcasted_iota(jnp.int32, (1, 4), 1)
    sel_ref[...] = jnp.where(k == 0, mi, jnp.where(k == 1, mj, 0))


def _tc_select(winp, winc):
    return pl.pallas_call(
        _tc_select_body,
        out_shape=jax.ShapeDtypeStruct((1, 4), jnp.int32),
    )(winp, winc)


def _tc_fin_body(sel_ref, winp_ref, winc_ref, prow_ref, p0_ref,
                 act_ref, lp_ref):
    mi = sel_ref[0]
    mj = sel_ref[1]
    big = jnp.int32(NN * NN)

    rows = lax.broadcasted_iota(jnp.int32, (NN, 1), 0)
    rmask = rows < mi
    rvals = jnp.where(rmask, winp_ref[...], NEG_INF)
    rbest = jnp.max(rvals)
    rrow = jnp.min(jnp.where(rvals == rbest, rows, big))
    rcol = jnp.sum(jnp.where(rows == rrow, winc_ref[...], 0))

    cols = lax.broadcasted_iota(jnp.int32, (1, NN), 1)
    r8 = lax.broadcasted_iota(jnp.int32, (8, NN), 0)
    pmi = jnp.sum(
        jnp.where(r8 == mi % 8, prow_ref[...], 0.0), axis=0,
        keepdims=True)
    mvals = jnp.where(cols < mj, pmi, NEG_INF)
    mbest = jnp.max(mvals)
    mcol = jnp.min(jnp.where(mvals == mbest, cols, big))

    use_rows = rbest >= mbest
    any_prefix = (rbest > NEG_INF) | (mbest > NEG_INF)
    idx_s = jnp.where(
        use_rows, rrow * NN + rcol, mi * NN + mcol)
    idx_s = jnp.where(any_prefix, idx_s, 0)
    idx_max = mi * NN + mj
    has_prefix = idx_max > 0
    si = jnp.where(has_prefix, idx_s // NN, 0)
    sj = jnp.where(has_prefix, idx_s % NN, 0)

    r2 = lax.broadcasted_iota(jnp.int32, (2, 2), 0)
    c2 = lax.broadcasted_iota(jnp.int32, (2, 2), 1)
    act = jnp.where(
        r2 == 0,
        jnp.where(c2 == 0, mi, mj),
        jnp.where(c2 == 0, si, sj))
    act_ref[...] = act

    p0 = jnp.sum(
        jnp.where(r8 == 0, p0_ref[...], 0.0), axis=0,
        keepdims=True)

    def gat(idx):
        return jnp.sum(jnp.where(cols == idx, p0, 0.0))

    lp = jnp.where(
        r2 == 0,
        jnp.where(c2 == 0, gat(mi), gat(mj)),
        jnp.where(c2 == 0, gat(si), gat(sj)))
    lp_ref[...] = -jnp.log(lp)


def _tc_finalize(sel, winp, winc, probs):
    grid_spec = pltpu.PrefetchScalarGridSpec(
        num_scalar_prefetch=1,
        grid=(1,),
        in_specs=[
            pl.BlockSpec((NN, 1), lambda i, s: (0, 0)),
            pl.BlockSpec((NN, 1), lambda i, s: (0, 0)),
            pl.BlockSpec((8, NN), lambda i, s: (s[0] // 8, 0)),
            pl.BlockSpec((8, NN), lambda i, s: (0, 0)),
        ],
        out_specs=[
            pl.BlockSpec((2, 2), lambda i, s: (0, 0)),
            pl.BlockSpec((2, 2), lambda i, s: (0, 0)),
        ],
    )
    return pl.pallas_call(
        _tc_fin_body,
        grid_spec=grid_spec,
        out_shape=(
            jax.ShapeDtypeStruct((2, 2), jnp.int32),
            jax.ShapeDtypeStruct((2, 2), jnp.float32),
        ),
    )(sel, winp, winc, probs, probs)


def kernel(x, edge_index, W1, b1, W2, b2, actor_W, actor_b, critic_W,
           critic_b):
    src = edge_index[0]
    dst = edge_index[1]

    A = _sc_abuild(src, dst)
    xa, crit = _tc_gcn(A, x, W1, b1, W2, b2, critic_W, critic_b)

    aw_t = jnp.transpose(actor_W)
    ab2 = actor_b.reshape(NN, NN)
    probs, winp, winc = _tc_actor(xa, aw_t, ab2)
    sel = _tc_select(winp, winc)
    actions, log_probs = _tc_finalize(sel.reshape(4), winp, winc, probs)
    critic = crit.reshape(1)
    return actions, log_probs, critic


# speedup vs baseline: 16.4960x; 1.1713x over previous
import functools

import jax
import jax.numpy as jnp
from jax import lax
from jax.experimental import pallas as pl
from jax.experimental.pallas import tpu as pltpu
from jax.experimental.pallas import tpu_sc as plsc

NN = 1024
NE = 65536
NW = 32
ROWS = NN // NW
SROWS = 2 * ROWS
ECH = 8192
NEG_INF = float("-inf")


def _wid():
    return lax.axis_index("s") * 2 + lax.axis_index("c")


def _sc_abuild_body(src_hbm, dst_hbm, out_hbm, src_v, dst_v, acc_v):
    w = _wid()
    half = w % 2
    slab = w // 2
    lo = slab * SROWS
    ebase = half * (NE // 2)

    def zero_body(i, _):
        for j in range(NN // 16):
            acc_v[i, pl.ds(j * 16, 16)] = jnp.zeros((16,), jnp.float32)
        return 0

    lax.fori_loop(0, SROWS, zero_body, 0)

    ones = jnp.ones((16,), jnp.float32)
    for k in range(NE // 2 // ECH):
        pltpu.sync_copy(src_hbm.at[pl.ds(ebase + k * ECH, ECH)], src_v)
        pltpu.sync_copy(dst_hbm.at[pl.ds(ebase + k * ECH, ECH)], dst_v)

        def group_body(g, _):
            for u in range(4):
                off = (g * 4 + u) * 16
                s16 = src_v[pl.ds(off, 16)]
                d16 = dst_v[pl.ds(off, 16)]
                local = d16 - lo
                mask = local.astype(jnp.uint32) < SROWS
                plsc.addupdate_scatter(acc_v, [local, s16], ones, mask=mask)
            return 0

        lax.fori_loop(0, ECH // 64, group_body, 0)

    pltpu.sync_copy(acc_v, out_hbm.at[half].at[pl.ds(lo, SROWS)])


def _sc_abuild(src, dst):
    mesh = plsc.VectorSubcoreMesh(core_axis_name="c", subcore_axis_name="s")
    f = pl.kernel(
        _sc_abuild_body,
        out_type=jax.ShapeDtypeStruct((2, NN, NN), jnp.float32),
        mesh=mesh,
        compiler_params=pltpu.CompilerParams(
            needs_layout_passes=False, use_tc_tiling_on_sc=False),
        scratch_types=[
            pltpu.VMEM((ECH,), jnp.int32),
            pltpu.VMEM((ECH,), jnp.int32),
            pltpu.VMEM((SROWS, NN), jnp.float32),
        ],
    )
    return f(src, dst)


def _tc_gcn_body(a_ref, x_ref, w1_ref, b1_ref, w2_ref, b2_ref, cw_ref,
                 cb_ref, xa_ref, crit_ref):
    A = a_ref[0] + a_ref[1]
    deg = jnp.sum(A, axis=1, keepdims=True) + 1.0
    dinv = lax.rsqrt(deg)

    xw1 = lax.dot_general(
        x_ref[...], w1_ref[...], (((1,), (1,)), ((), ())),
        preferred_element_type=jnp.float32)
    t1 = dinv * xw1
    a1 = lax.dot_general(
        A, t1, (((1,), (0,)), ((), ())), preferred_element_type=jnp.float32)
    h1 = jnp.maximum(dinv * (a1 + t1) + b1_ref[...][None, :], 0.0)

    h1b = h1.astype(jnp.bfloat16).astype(jnp.float32)
    xw2 = lax.dot_general(
        h1b, w2_ref[...], (((1,), (1,)), ((), ())),
        preferred_element_type=jnp.float32)
    t2 = dinv * xw2
    a2 = lax.dot_general(
        A, t2, (((1,), (0,)), ((), ())), preferred_element_type=jnp.float32)
    h2 = jnp.maximum(dinv * (a2 + t2) + b2_ref[...][None, :], 0.0)

    xa_ref[...] = jnp.sum(h2, axis=0, keepdims=True) / 1024.0
    h2b = h2.astype(jnp.bfloat16).astype(jnp.float32)
    colsum = jnp.sum(h2b, axis=0, keepdims=True)
    total = jnp.sum(colsum * cw_ref[...])
    crit_ref[...] = jnp.broadcast_to(total / 1024.0 + cb_ref[0], (1, 1))


def _tc_gcn(A, x, W1, b1, W2, b2, critic_W, critic_b):
    return pl.pallas_call(
        _tc_gcn_body,
        out_shape=(
            jax.ShapeDtypeStruct((1, 20), jnp.float32),
            jax.ShapeDtypeStruct((1, 1), jnp.float32),
        ),
    )(A, x, W1, b1, W2, b2, critic_W, critic_b)


_BR = 128


def _tc_actor_body(xa_ref, aw_ref, ab_ref, probs_ref, winp_ref, winc_ref):
    l = lax.dot_general(
        xa_ref[...], aw_ref[...], (((1,), (0,)), ((), ())),
        preferred_element_type=jnp.float32)
    acc = jnp.reshape(l, (_BR, NN)) + ab_ref[...]
    m = jnp.max(acc, axis=1, keepdims=True)
    e = jnp.exp(acc - m)
    s = jnp.sum(e, axis=1, keepdims=True)
    p = e / s
    probs_ref[...] = p
    wp = jnp.max(p, axis=1, keepdims=True)
    winp_ref[...] = wp
    cols = lax.broadcasted_iota(jnp.int32, (_BR, NN), 1)
    winc_ref[...] = jnp.min(
        jnp.where(p == wp, cols, jnp.int32(NN * NN)), axis=1, keepdims=True)


def _tc_actor(xa, aw3, ab2):
    grid = (NN // _BR,)
    return pl.pallas_call(
        _tc_actor_body,
        grid=grid,
        in_specs=[
            pl.BlockSpec((1, 20), lambda i: (0, 0)),
            pl.BlockSpec((20, _BR * NN), lambda i: (0, i)),
            pl.BlockSpec((_BR, NN), lambda i: (i, 0)),
        ],
        out_specs=[
            pl.BlockSpec((_BR, NN), lambda i: (i, 0)),
            pl.BlockSpec((_BR, 1), lambda i: (i, 0)),
            pl.BlockSpec((_BR, 1), lambda i: (i, 0)),
        ],
        out_shape=(
            jax.ShapeDtypeStruct((NN, NN), jnp.float32),
            jax.ShapeDtypeStruct((NN, 1), jnp.float32),
            jax.ShapeDtypeStruct((NN, 1), jnp.int32),
        ),
    )(xa, aw3, ab2)


def _tc_select_body(winp_ref, winc_ref, sel_ref):
    wp = winp_ref[...]
    rows = lax.broadcasted_iota(jnp.int32, (NN, 1), 0)
    maxv = jnp.max(wp)
    mi = jnp.min(jnp.where(wp == maxv, rows, jnp.int32(NN * NN)))
    mj = jnp.sum(jnp.where(rows == mi, winc_ref[...], 0))
    k = lax.broadcasted_iota(jnp.int32, (1, 4), 1)
    sel_ref[...] = jnp.where(k == 0, mi, jnp.where(k == 1, mj, 0))


def _tc_select(winp, winc):
    return pl.pallas_call(
        _tc_select_body,
        out_shape=jax.ShapeDtypeStruct((1, 4), jnp.int32),
    )(winp, winc)


def _tc_fin_body(sel_ref, winp_ref, winc_ref, prow_ref, p0_ref,
                 act_ref, lp_ref):
    mi = sel_ref[0]
    mj = sel_ref[1]
    big = jnp.int32(NN * NN)

    rows = lax.broadcasted_iota(jnp.int32, (NN, 1), 0)
    rmask = rows < mi
    rvals = jnp.where(rmask, winp_ref[...], NEG_INF)
    rbest = jnp.max(rvals)
    rrow = jnp.min(jnp.where(rvals == rbest, rows, big))
    rcol = jnp.sum(jnp.where(rows == rrow, winc_ref[...], 0))

    cols = lax.broadcasted_iota(jnp.int32, (1, NN), 1)
    r8 = lax.broadcasted_iota(jnp.int32, (8, NN), 0)
    pmi = jnp.sum(
        jnp.where(r8 == mi % 8, prow_ref[...], 0.0), axis=0,
        keepdims=True)
    mvals = jnp.where(cols < mj, pmi, NEG_INF)
    mbest = jnp.max(mvals)
    mcol = jnp.min(jnp.where(mvals == mbest, cols, big))

    use_rows = rbest >= mbest
    any_prefix = (rbest > NEG_INF) | (mbest > NEG_INF)
    idx_s = jnp.where(
        use_rows, rrow * NN + rcol, mi * NN + mcol)
    idx_s = jnp.where(any_prefix, idx_s, 0)
    idx_max = mi * NN + mj
    has_prefix = idx_max > 0
    si = jnp.where(has_prefix, idx_s // NN, 0)
    sj = jnp.where(has_prefix, idx_s % NN, 0)

    r2 = lax.broadcasted_iota(jnp.int32, (2, 2), 0)
    c2 = lax.broadcasted_iota(jnp.int32, (2, 2), 1)
    act = jnp.where(
        r2 == 0,
        jnp.where(c2 == 0, mi, mj),
        jnp.where(c2 == 0, si, sj))
    act_ref[...] = act

    p0 = jnp.sum(
        jnp.where(r8 == 0, p0_ref[...], 0.0), axis=0,
        keepdims=True)

    def gat(idx):
        return jnp.sum(jnp.where(cols == idx, p0, 0.0))

    lp = jnp.where(
        r2 == 0,
        jnp.where(c2 == 0, gat(mi), gat(mj)),
        jnp.where(c2 == 0, gat(si), gat(sj)))
    lp_ref[...] = -jnp.log(lp)


def _tc_finalize(sel, winp, winc, probs):
    grid_spec = pltpu.PrefetchScalarGridSpec(
        num_scalar_prefetch=1,
        grid=(1,),
        in_specs=[
            pl.BlockSpec((NN, 1), lambda i, s: (0, 0)),
            pl.BlockSpec((NN, 1), lambda i, s: (0, 0)),
            pl.BlockSpec((8, NN), lambda i, s: (s[0] // 8, 0)),
            pl.BlockSpec((8, NN), lambda i, s: (0, 0)),
        ],
        out_specs=[
            pl.BlockSpec((2, 2), lambda i, s: (0, 0)),
            pl.BlockSpec((2, 2), lambda i, s: (0, 0)),
        ],
    )
    return pl.pallas_call(
        _tc_fin_body,
        grid_spec=grid_spec,
        out_shape=(
            jax.ShapeDtypeStruct((2, 2), jnp.int32),
            jax.ShapeDtypeStruct((2, 2), jnp.float32),
        ),
    )(sel, winp, winc, probs, probs)


def kernel(x, edge_index, W1, b1, W2, b2, actor_W, actor_b, critic_W,
           critic_b):
    src = edge_index[0]
    dst = edge_index[1]

    A = _sc_abuild(src, dst)
    xa, crit = _tc_gcn(A, x, W1, b1, W2, b2, critic_W, critic_b)

    aw_t = jnp.transpose(actor_W)
    ab2 = actor_b.reshape(NN, NN)
    probs, winp, winc = _tc_actor(xa, aw_t, ab2)
    sel = _tc_select(winp, winc)
    actions, log_probs = _tc_finalize(sel.reshape(4), winp, winc, probs)
    critic = crit.reshape(1)
    return actions, log_probs, critic
